# baseline (speedup 1.0000x reference)
import jax
import jax.numpy as jnp
from jax import lax
from jax.experimental import pallas as pl
from jax.experimental.pallas import tpu as pltpu
from jax.experimental.pallas import tpu_sc as plsc

N_NODES = 10000
N_PAD = 10240
N_EDGES = 320000
NC = 2
NS = 16
EDGES_PER_TILE = N_EDGES // (NC * NS)
CHUNK = 2000
N_CHUNKS = EDGES_PER_TILE // CHUNK
GCHUNK = 1000
N_GCHUNKS = EDGES_PER_TILE // GCHUNK
ROWS_PER_TILE = N_PAD // NS

_mesh = plsc.VectorSubcoreMesh(core_axis_name="c", subcore_axis_name="s")
_sc_params = pltpu.CompilerParams(use_tc_tiling_on_sc=False)


def _sc_degrees_body(src_hbm, dst_hbm, ones_hbm, zeros1_hbm,
                     degs_hbm, degd_hbm,
                     hist_s, hist_d, ibs, ibd, ones_v, obuf,
                     isem, ssem):
    c = lax.axis_index("c")
    s = lax.axis_index("s")
    wid = s * NC + c
    r0 = s * ROWS_PER_TILE
    base = wid * EDGES_PER_TILE
    for k in range(N_CHUNKS):
        off = base + k * CHUNK
        pltpu.async_copy(src_hbm.at[pl.ds(off, CHUNK)], ibs.at[k], isem)
        pltpu.async_copy(dst_hbm.at[pl.ds(off, CHUNK)], ibd.at[k], isem)
    pltpu.sync_copy(ones_hbm, ones_v)
    pltpu.sync_copy(zeros1_hbm.at[pl.ds(r0, ROWS_PER_TILE)],
                    hist_s.at[pl.ds(r0, ROWS_PER_TILE)])
    pltpu.sync_copy(zeros1_hbm.at[pl.ds(r0, ROWS_PER_TILE)],
                    hist_d.at[pl.ds(r0, ROWS_PER_TILE)])
    for k in range(N_CHUNKS):
        off = base + k * CHUNK
        pltpu.make_async_copy(src_hbm.at[pl.ds(off, CHUNK)],
                              ibs.at[k], isem).wait()
        pltpu.make_async_copy(dst_hbm.at[pl.ds(off, CHUNK)],
                              ibd.at[k], isem).wait()
    plsc.subcore_barrier()
    for k in range(N_CHUNKS):
        pltpu.async_copy(ones_v, hist_s.at[ibs.at[k]], ssem, add=True)
        pltpu.async_copy(ones_v, hist_d.at[ibd.at[k]], ssem, add=True)
    for k in range(N_CHUNKS):
        pltpu.make_async_copy(ones_v, hist_s.at[ibs.at[k]], ssem).wait()
        pltpu.make_async_copy(ones_v, hist_d.at[ibd.at[k]], ssem).wait()
    plsc.subcore_barrier()
    pltpu.sync_copy(hist_s.at[pl.ds(r0, ROWS_PER_TILE)], obuf)
    pltpu.sync_copy(obuf, degs_hbm.at[c, pl.ds(r0, ROWS_PER_TILE)])
    pltpu.sync_copy(hist_d.at[pl.ds(r0, ROWS_PER_TILE)], obuf)
    pltpu.sync_copy(obuf, degd_hbm.at[c, pl.ds(r0, ROWS_PER_TILE)])


def _sc_degrees(src, dst, ones, zeros1):
    fn = pl.kernel(
        _sc_degrees_body,
        out_type=(jax.ShapeDtypeStruct((NC, N_PAD), jnp.float32),
                  jax.ShapeDtypeStruct((NC, N_PAD), jnp.float32)),
        mesh=_mesh,
        compiler_params=_sc_params,
        scratch_types=(
            pltpu.VMEM_SHARED((N_PAD,), jnp.float32),
            pltpu.VMEM_SHARED((N_PAD,), jnp.float32),
            pltpu.VMEM((N_CHUNKS, CHUNK), jnp.int32),
            pltpu.VMEM((N_CHUNKS, CHUNK), jnp.int32),
            pltpu.VMEM((CHUNK,), jnp.float32),
            pltpu.VMEM((ROWS_PER_TILE,), jnp.float32),
            pltpu.SemaphoreType.DMA,
            pltpu.SemaphoreType.DMA,
        ),
    )
    return fn(src, dst, ones, zeros1)


def _sc_agg_body(tab_hbm, src_hbm, dst_hbm, zeros2_hbm,
                 out_hbm,
                 agg_sh, ibs, ibd, rb0, rb1, obuf,
                 isem, gsem, ss0, ss1):
    c = lax.axis_index("c")
    s = lax.axis_index("s")
    wid = s * NC + c
    r0 = s * ROWS_PER_TILE
    base = wid * EDGES_PER_TILE
    for k in range(N_CHUNKS):
        off = base + k * CHUNK
        pltpu.async_copy(src_hbm.at[pl.ds(off, CHUNK)], ibs.at[k], isem)
        pltpu.async_copy(dst_hbm.at[pl.ds(off, CHUNK)], ibd.at[k], isem)
    pltpu.sync_copy(zeros2_hbm.at[pl.ds(r0, ROWS_PER_TILE)],
                    agg_sh.at[pl.ds(r0, ROWS_PER_TILE)])
    for k in range(N_CHUNKS):
        off = base + k * CHUNK
        pltpu.make_async_copy(src_hbm.at[pl.ds(off, CHUNK)],
                              ibs.at[k], isem).wait()
        pltpu.make_async_copy(dst_hbm.at[pl.ds(off, CHUNK)],
                              ibd.at[k], isem).wait()
    plsc.subcore_barrier()
    rbs = (rb0, rb1)
    sss = (ss0, ss1)
    for k in range(N_CHUNKS):
        b = k % 2
        if k >= 2:
            pltpu.make_async_copy(rbs[b], agg_sh.at[ibd.at[k - 2]],
                                  sss[b]).wait()
        pltpu.async_copy(tab_hbm.at[ibs.at[k]], rbs[b], gsem)
        pltpu.make_async_copy(tab_hbm.at[ibs.at[k]], rbs[b], gsem).wait()
        pltpu.async_copy(rbs[b], agg_sh.at[ibd.at[k]], sss[b], add=True)
    for k in range(N_CHUNKS - 2, N_CHUNKS):
        b = k % 2
        pltpu.make_async_copy(rbs[b], agg_sh.at[ibd.at[k]], sss[b]).wait()
    plsc.subcore_barrier()
    pltpu.sync_copy(agg_sh.at[pl.ds(r0, ROWS_PER_TILE)], obuf)
    pltpu.sync_copy(obuf, out_hbm.at[c, pl.ds(r0, ROWS_PER_TILE)])


def _sc_agg(table, src, dst, zeros2):
    fn = pl.kernel(
        _sc_agg_body,
        out_type=jax.ShapeDtypeStruct((NC, N_PAD, 16), jnp.float32),
        mesh=_mesh,
        compiler_params=_sc_params,
        scratch_types=(
            pltpu.VMEM_SHARED((N_PAD, 16), jnp.float32),
            pltpu.VMEM((N_CHUNKS, CHUNK), jnp.int32),
            pltpu.VMEM((N_CHUNKS, CHUNK), jnp.int32),
            pltpu.VMEM((CHUNK, 16), jnp.float32),
            pltpu.VMEM((CHUNK, 16), jnp.float32),
            pltpu.VMEM((ROWS_PER_TILE, 16), jnp.float32),
            pltpu.SemaphoreType.DMA,
            pltpu.SemaphoreType.DMA,
            pltpu.SemaphoreType.DMA,
            pltpu.SemaphoreType.DMA,
        ),
    )
    return fn(table, src, dst, zeros2)


_RING = 3


def _sc_edge_gather_body(a_hbm, b_hbm, src_hbm, dst_hbm,
                         g_hbm,
                         ibs, ibd, ra0, ra1, ra2, rb0, rb1, rb2,
                         isem, ga0, ga1, ga2, gb0, gb1, gb2,
                         sa0, sa1, sa2):
    c = lax.axis_index("c")
    s = lax.axis_index("s")
    wid = s * NC + c
    base = wid * EDGES_PER_TILE
    for k in range(N_GCHUNKS):
        off = base + k * GCHUNK
        pltpu.async_copy(src_hbm.at[pl.ds(off, GCHUNK)], ibs.at[k], isem)
        pltpu.async_copy(dst_hbm.at[pl.ds(off, GCHUNK)], ibd.at[k], isem)
    for k in range(N_GCHUNKS):
        off = base + k * GCHUNK
        pltpu.make_async_copy(src_hbm.at[pl.ds(off, GCHUNK)],
                              ibs.at[k], isem).wait()
        pltpu.make_async_copy(dst_hbm.at[pl.ds(off, GCHUNK)],
                              ibd.at[k], isem).wait()
    ras = (ra0, ra1, ra2)
    rbs = (rb0, rb1, rb2)
    gsa = (ga0, ga1, ga2)
    gsb = (gb0, gb1, gb2)
    sas = (sa0, sa1, sa2)
    for k in range(min(_RING, N_GCHUNKS)):
        b = k % _RING
        pltpu.async_copy(a_hbm.at[ibs.at[k]], ras[b], gsa[b])
        pltpu.async_copy(b_hbm.at[ibd.at[k]], rbs[b], gsb[b])
    for k in range(N_GCHUNKS):
        b = k % _RING
        off = base + k * GCHUNK
        pltpu.make_async_copy(a_hbm.at[ibs.at[k]], ras[b], gsa[b]).wait()
        pltpu.make_async_copy(b_hbm.at[ibd.at[k]], rbs[b], gsb[b]).wait()

        def add_row(i, carry, ra=ras[b], rb=rbs[b]):
            ra[i, :] = ra[i, :] + rb[i, :]
            return carry

        lax.fori_loop(0, GCHUNK, add_row, 0, unroll=4)
        pltpu.async_copy(ras[b],
                         g_hbm.at[pl.ds(off, GCHUNK), pl.ds(0, 16)], sas[b])
        nk = k + _RING
        if nk < N_GCHUNKS:
            poff = base + k * GCHUNK
            pltpu.make_async_copy(
                ras[b], g_hbm.at[pl.ds(poff, GCHUNK), pl.ds(0, 16)],
                sas[b]).wait()
            pltpu.async_copy(a_hbm.at[ibs.at[nk]], ras[b], gsa[b])
            pltpu.async_copy(b_hbm.at[ibd.at[nk]], rbs[b], gsb[b])
    for k in range(N_GCHUNKS - min(_RING, N_GCHUNKS), N_GCHUNKS):
        b = k % _RING
        off = base + k * GCHUNK
        pltpu.make_async_copy(
            ras[b], g_hbm.at[pl.ds(off, GCHUNK), pl.ds(0, 16)],
            sas[b]).wait()


def _sc_edge_gather(a_tab, b_tab, src, dst):
    fn = pl.kernel(
        _sc_edge_gather_body,
        out_type=jax.ShapeDtypeStruct((N_EDGES, 128), jnp.float32),
        mesh=_mesh,
        compiler_params=_sc_params,
        scratch_types=(
            pltpu.VMEM((N_GCHUNKS, GCHUNK), jnp.int32),
            pltpu.VMEM((N_GCHUNKS, GCHUNK), jnp.int32),
            pltpu.VMEM((GCHUNK, 16), jnp.float32),
            pltpu.VMEM((GCHUNK, 16), jnp.float32),
            pltpu.VMEM((GCHUNK, 16), jnp.float32),
            pltpu.VMEM((GCHUNK, 16), jnp.float32),
            pltpu.VMEM((GCHUNK, 16), jnp.float32),
            pltpu.VMEM((GCHUNK, 16), jnp.float32),
            pltpu.SemaphoreType.DMA,
            pltpu.SemaphoreType.DMA,
            pltpu.SemaphoreType.DMA,
            pltpu.SemaphoreType.DMA,
            pltpu.SemaphoreType.DMA,
            pltpu.SemaphoreType.DMA,
            pltpu.SemaphoreType.DMA,
            pltpu.SemaphoreType.DMA,
            pltpu.SemaphoreType.DMA,
            pltpu.SemaphoreType.DMA,
        ),
    )
    return fn(a_tab, b_tab, src, dst)


def _tc_h1_body(nf_ref, w1_ref, degs_ref, degd_ref,
                h1p_ref, pq_ref):
    ds = degs_ref[...]
    dd = degd_ref[...]
    degs = ds[0, :N_NODES] + ds[1, :N_NODES]
    degd = dd[0, :N_NODES] + dd[1, :N_NODES]
    p = lax.rsqrt(jnp.maximum(degs, 1.0))
    q = lax.rsqrt(jnp.maximum(degd, 1.0))
    h1 = jnp.dot(nf_ref[...], w1_ref[...], preferred_element_type=jnp.float32)
    h1p_ref[...] = h1 * p[:, None]
    pq_ref[...] = jnp.concatenate([p[:, None], q[:, None]], axis=1)


def _tc_h1(n_feats, W1, degs, degd):
    return pl.pallas_call(
        _tc_h1_body,
        out_shape=(jax.ShapeDtypeStruct((N_NODES, 16), jnp.float32),
                   jax.ShapeDtypeStruct((N_NODES, 2), jnp.float32)),
    )(n_feats, W1, degs, degd)


def _tc_mid_body(p_ref, pq_ref, b1_ref, g1_ref, be1_ref, y_ref):
    parr = p_ref[...]
    agg = parr[0, :N_NODES, :] + parr[1, :N_NODES, :]
    q = pq_ref[:, 1]
    p = pq_ref[:, 0]
    s1 = g1_ref[...] * (1.0 / jnp.sqrt(1.0 + 1e-5))
    x1 = jnp.maximum(agg * q[:, None] + b1_ref[...][None, :], 0.0)
    y_ref[...] = (x1 * s1[None, :] + be1_ref[...][None, :]) * p[:, None]


def _tc_mid(partials, pq, b1, g1, be1):
    return pl.pallas_call(
        _tc_mid_body,
        out_shape=jax.ShapeDtypeStruct((N_NODES, 16), jnp.float32),
    )(partials, pq, b1, g1, be1)


def _tc_ab_body(p_ref, pq_ref, w2_ref, b2_ref, wp1_ref, sp_ref, wp2_ref,
                a_ref, b_ref):
    parr = p_ref[...]
    agg = parr[0, :N_NODES, :] + parr[1, :N_NODES, :]
    q = pq_ref[:, 1]
    aggq = agg * q[:, None]
    wp1 = wp1_ref[...]
    sp = sp_ref[...] * (1.0 / jnp.sqrt(1.0 + 1e-5))
    w2p = sp[:, None] * wp2_ref[...]
    ms = jnp.dot(wp1[16:80, :], w2p, preferred_element_type=jnp.float32)
    md = jnp.dot(wp1[80:144, :], w2p, preferred_element_type=jnp.float32)
    wa = jnp.dot(w2_ref[...], ms, preferred_element_type=jnp.float32)
    wb = jnp.dot(w2_ref[...], md, preferred_element_type=jnp.float32)
    ca = jnp.dot(b2_ref[...][None, :], ms, preferred_element_type=jnp.float32)
    cb = jnp.dot(b2_ref[...][None, :], md, preferred_element_type=jnp.float32)
    a = jnp.dot(aggq, wa, preferred_element_type=jnp.float32) + ca
    b = jnp.dot(aggq, wb, preferred_element_type=jnp.float32) + cb
    zpad = jnp.zeros((N_NODES, 6), dtype=jnp.float32)
    a_ref[...] = jnp.concatenate([a, zpad], axis=1)
    b_ref[...] = jnp.concatenate([b, zpad], axis=1)


def _tc_ab(partials, pq, W2, b2, Wp1, sp, Wp2):
    return pl.pallas_call(
        _tc_ab_body,
        out_shape=(jax.ShapeDtypeStruct((N_NODES, 16), jnp.float32),
                   jax.ShapeDtypeStruct((N_NODES, 16), jnp.float32)),
    )(partials, pq, W2, b2, Wp1, sp, Wp2)


_EBLK = 6400


def _tc_final_body(eft_ref, g_ref, wp1et_ref, wp2t_ref,
                   gp_ref, bepc_ref, bp1c_ref, bp2c_ref, out_ref):
    sp = gp_ref[...][0] * (1.0 / jnp.sqrt(1.0 + 1e-5))
    w2pt = wp2t_ref[...] * sp[None, :]
    met = jnp.dot(w2pt, wp1et_ref[...],
                  preferred_element_type=jnp.float32)
    c0t = (jnp.dot(w2pt, bp1c_ref[...],
                   preferred_element_type=jnp.float32)
           + jnp.dot(wp2t_ref[...], bepc_ref[...],
                     preferred_element_type=jnp.float32)
           + bp2c_ref[...])
    c0pt = jnp.concatenate(
        [c0t, jnp.full((6, 1), -1e30, jnp.float32)], axis=0)
    mept = jnp.concatenate(
        [met, jnp.zeros((6, 16), jnp.float32)], axis=0)
    zt = jnp.dot(mept, eft_ref[...], preferred_element_type=jnp.float32)
    gt = jnp.transpose(g_ref[...][:, :16])
    zt = zt + gt + c0pt
    m = jnp.max(zt, axis=0, keepdims=True)
    e = jnp.exp(zt - m)
    lse = jnp.log(jnp.sum(e, axis=0, keepdims=True))
    out_ref[...] = (zt - m - lse)[:10, :]


def _tc_final(eft, g, Wp1eT, Wp2T, gp, bepc, bp1c, bp2c):
    grid = N_EDGES // _EBLK
    return pl.pallas_call(
        _tc_final_body,
        grid=(grid,),
        in_specs=[
            pl.BlockSpec((16, _EBLK), lambda i: (0, i)),
            pl.BlockSpec((_EBLK, 128), lambda i: (i, 0)),
            pl.BlockSpec((16, 16), lambda i: (0, 0)),
            pl.BlockSpec((10, 16), lambda i: (0, 0)),
            pl.BlockSpec((1, 16), lambda i: (0, 0)),
            pl.BlockSpec((16, 1), lambda i: (0, 0)),
            pl.BlockSpec((16, 1), lambda i: (0, 0)),
            pl.BlockSpec((10, 1), lambda i: (0, 0)),
        ],
        out_specs=pl.BlockSpec((10, _EBLK), lambda i: (0, i)),
        out_shape=jax.ShapeDtypeStruct((10, N_EDGES), jnp.float32),
    )(eft, g, Wp1eT, Wp2T, gp, bepc, bp1c, bp2c)


@jax.jit
def _run(n_feats, edge_index, edge_feat, W1, b1, bn1_gamma, bn1_beta,
         W2, b2, Wp1, bp1, bnp_gamma, bnp_beta, Wp2, bp2):
    src = edge_index[0]
    dst = edge_index[1]
    ones = jnp.ones((CHUNK,), jnp.float32)
    zeros1 = jnp.zeros((N_PAD,), jnp.float32)
    zeros2 = jnp.zeros((N_PAD, 16), jnp.float32)

    degs, degd = _sc_degrees(src, dst, ones, zeros1)
    h1p, pq = _tc_h1(n_feats, W1, degs, degd)
    agg1 = _sc_agg(h1p, src, dst, zeros2)
    y = _tc_mid(agg1, pq, b1, bn1_gamma, bn1_beta)
    agg2 = _sc_agg(y, src, dst, zeros2)
    a_tab, b_tab = _tc_ab(agg2, pq, W2, b2, Wp1, bnp_gamma, Wp2)
    g = _sc_edge_gather(a_tab, b_tab, src, dst)
    outt = _tc_final(edge_feat.T, g, Wp1[:16, :].T, Wp2.T,
                     bnp_gamma[None, :], bnp_beta[:, None],
                     bp1[:, None], bp2[:, None])
    return outt.T


def kernel(n_feats, edge_index, edge_feat, W1, b1, bn1_gamma, bn1_beta,
           W2, b2, Wp1, bp1, bnp_gamma, bnp_beta, Wp2, bp2):
    return _run(n_feats, edge_index, edge_feat, W1, b1, bn1_gamma, bn1_beta,
                W2, b2, Wp1, bp1, bnp_gamma, bnp_beta, Wp2, bp2)


# speedup vs baseline: 1.0549x; 1.0549x over previous
import jax
import jax.numpy as jnp
from jax import lax
from jax.experimental import pallas as pl
from jax.experimental.pallas import tpu as pltpu
from jax.experimental.pallas import tpu_sc as plsc

N_NODES = 10000
N_PAD = 10240
N_EDGES = 320000
NC = 2
NS = 16
EDGES_PER_TILE = N_EDGES // (NC * NS)
CHUNK = 2000
N_CHUNKS = EDGES_PER_TILE // CHUNK
GCHUNK = 1000
N_GCHUNKS = EDGES_PER_TILE // GCHUNK
ROWS_PER_TILE = N_PAD // NS

_mesh = plsc.VectorSubcoreMesh(core_axis_name="c", subcore_axis_name="s")
_sc_params = pltpu.CompilerParams(use_tc_tiling_on_sc=False)


def _sc_degrees_body(src_hbm, dst_hbm, ones_hbm, zeros1_hbm,
                     degs_hbm, degd_hbm,
                     hist_s, hist_d, ibs, ibd, ones_v, obuf,
                     isem, ssem):
    c = lax.axis_index("c")
    s = lax.axis_index("s")
    wid = s * NC + c
    r0 = s * ROWS_PER_TILE
    base = wid * EDGES_PER_TILE
    for k in range(N_CHUNKS):
        off = base + k * CHUNK
        pltpu.async_copy(src_hbm.at[pl.ds(off, CHUNK)], ibs.at[k], isem)
        pltpu.async_copy(dst_hbm.at[pl.ds(off, CHUNK)], ibd.at[k], isem)
    pltpu.sync_copy(ones_hbm, ones_v)
    pltpu.sync_copy(zeros1_hbm.at[pl.ds(r0, ROWS_PER_TILE)],
                    hist_s.at[pl.ds(r0, ROWS_PER_TILE)])
    pltpu.sync_copy(zeros1_hbm.at[pl.ds(r0, ROWS_PER_TILE)],
                    hist_d.at[pl.ds(r0, ROWS_PER_TILE)])
    for k in range(N_CHUNKS):
        off = base + k * CHUNK
        pltpu.make_async_copy(src_hbm.at[pl.ds(off, CHUNK)],
                              ibs.at[k], isem).wait()
        pltpu.make_async_copy(dst_hbm.at[pl.ds(off, CHUNK)],
                              ibd.at[k], isem).wait()
    plsc.subcore_barrier()
    for k in range(N_CHUNKS):
        pltpu.async_copy(ones_v, hist_s.at[ibs.at[k]], ssem, add=True)
        pltpu.async_copy(ones_v, hist_d.at[ibd.at[k]], ssem, add=True)
    for k in range(N_CHUNKS):
        pltpu.make_async_copy(ones_v, hist_s.at[ibs.at[k]], ssem).wait()
        pltpu.make_async_copy(ones_v, hist_d.at[ibd.at[k]], ssem).wait()
    plsc.subcore_barrier()
    pltpu.sync_copy(hist_s.at[pl.ds(r0, ROWS_PER_TILE)], obuf)
    pltpu.sync_copy(obuf, degs_hbm.at[c, pl.ds(r0, ROWS_PER_TILE)])
    pltpu.sync_copy(hist_d.at[pl.ds(r0, ROWS_PER_TILE)], obuf)
    pltpu.sync_copy(obuf, degd_hbm.at[c, pl.ds(r0, ROWS_PER_TILE)])


def _sc_degrees(src, dst, ones, zeros1):
    fn = pl.kernel(
        _sc_degrees_body,
        out_type=(jax.ShapeDtypeStruct((NC, N_PAD), jnp.float32),
                  jax.ShapeDtypeStruct((NC, N_PAD), jnp.float32)),
        mesh=_mesh,
        compiler_params=_sc_params,
        scratch_types=(
            pltpu.VMEM_SHARED((N_PAD,), jnp.float32),
            pltpu.VMEM_SHARED((N_PAD,), jnp.float32),
            pltpu.VMEM((N_CHUNKS, CHUNK), jnp.int32),
            pltpu.VMEM((N_CHUNKS, CHUNK), jnp.int32),
            pltpu.VMEM((CHUNK,), jnp.float32),
            pltpu.VMEM((ROWS_PER_TILE,), jnp.float32),
            pltpu.SemaphoreType.DMA,
            pltpu.SemaphoreType.DMA,
        ),
    )
    return fn(src, dst, ones, zeros1)


def _sc_agg_body(tab_hbm, src_hbm, dst_hbm, zeros2_hbm,
                 out_hbm,
                 agg_sh, ibs, ibd, rb0, rb1, obuf,
                 isem, gsem, ss0, ss1):
    c = lax.axis_index("c")
    s = lax.axis_index("s")
    wid = s * NC + c
    r0 = s * ROWS_PER_TILE
    base = wid * EDGES_PER_TILE
    for k in range(N_CHUNKS):
        off = base + k * CHUNK
        pltpu.async_copy(src_hbm.at[pl.ds(off, CHUNK)], ibs.at[k], isem)
        pltpu.async_copy(dst_hbm.at[pl.ds(off, CHUNK)], ibd.at[k], isem)
    pltpu.sync_copy(zeros2_hbm.at[pl.ds(r0, ROWS_PER_TILE)],
                    agg_sh.at[pl.ds(r0, ROWS_PER_TILE)])
    for k in range(N_CHUNKS):
        off = base + k * CHUNK
        pltpu.make_async_copy(src_hbm.at[pl.ds(off, CHUNK)],
                              ibs.at[k], isem).wait()
        pltpu.make_async_copy(dst_hbm.at[pl.ds(off, CHUNK)],
                              ibd.at[k], isem).wait()
    plsc.subcore_barrier()
    rbs = (rb0, rb1)
    sss = (ss0, ss1)
    for k in range(N_CHUNKS):
        b = k % 2
        if k >= 2:
            pltpu.make_async_copy(rbs[b], agg_sh.at[ibd.at[k - 2]],
                                  sss[b]).wait()
        pltpu.async_copy(tab_hbm.at[ibs.at[k]], rbs[b], gsem)
        pltpu.make_async_copy(tab_hbm.at[ibs.at[k]], rbs[b], gsem).wait()
        pltpu.async_copy(rbs[b], agg_sh.at[ibd.at[k]], sss[b], add=True)
    for k in range(N_CHUNKS - 2, N_CHUNKS):
        b = k % 2
        pltpu.make_async_copy(rbs[b], agg_sh.at[ibd.at[k]], sss[b]).wait()
    plsc.subcore_barrier()
    pltpu.sync_copy(agg_sh.at[pl.ds(r0, ROWS_PER_TILE)], obuf)
    pltpu.sync_copy(obuf, out_hbm.at[c, pl.ds(r0, ROWS_PER_TILE)])


def _sc_agg(table, src, dst, zeros2):
    fn = pl.kernel(
        _sc_agg_body,
        out_type=jax.ShapeDtypeStruct((NC, N_PAD, 16), jnp.float32),
        mesh=_mesh,
        compiler_params=_sc_params,
        scratch_types=(
            pltpu.VMEM_SHARED((N_PAD, 16), jnp.float32),
            pltpu.VMEM((N_CHUNKS, CHUNK), jnp.int32),
            pltpu.VMEM((N_CHUNKS, CHUNK), jnp.int32),
            pltpu.VMEM((CHUNK, 16), jnp.float32),
            pltpu.VMEM((CHUNK, 16), jnp.float32),
            pltpu.VMEM((ROWS_PER_TILE, 16), jnp.float32),
            pltpu.SemaphoreType.DMA,
            pltpu.SemaphoreType.DMA,
            pltpu.SemaphoreType.DMA,
            pltpu.SemaphoreType.DMA,
        ),
    )
    return fn(table, src, dst, zeros2)


def _sc_edge_gather_body(a_hbm, b_hbm, src_hbm, dst_hbm,
                         g_hbm,
                         ibs, ibd, ra0, ra1, rb0, rb1,
                         isem, gsa, gsb, sa0, sa1):
    c = lax.axis_index("c")
    s = lax.axis_index("s")
    wid = s * NC + c
    base = wid * EDGES_PER_TILE
    for k in range(N_GCHUNKS):
        off = base + k * GCHUNK
        pltpu.async_copy(src_hbm.at[pl.ds(off, GCHUNK)], ibs.at[k], isem)
        pltpu.async_copy(dst_hbm.at[pl.ds(off, GCHUNK)], ibd.at[k], isem)
    for k in range(N_GCHUNKS):
        off = base + k * GCHUNK
        pltpu.make_async_copy(src_hbm.at[pl.ds(off, GCHUNK)],
                              ibs.at[k], isem).wait()
        pltpu.make_async_copy(dst_hbm.at[pl.ds(off, GCHUNK)],
                              ibd.at[k], isem).wait()
    ras = (ra0, ra1)
    rbs = (rb0, rb1)
    sas = (sa0, sa1)
    pltpu.async_copy(a_hbm.at[ibs.at[0]], ras[0], gsa)
    pltpu.async_copy(b_hbm.at[ibd.at[0]], rbs[0], gsb)
    for k in range(N_GCHUNKS):
        b = k % 2
        nb = (k + 1) % 2
        off = base + k * GCHUNK
        pltpu.make_async_copy(a_hbm.at[ibs.at[k]], ras[b], gsa).wait()
        pltpu.make_async_copy(b_hbm.at[ibd.at[k]], rbs[b], gsb).wait()
        if k + 1 < N_GCHUNKS:
            if k >= 1:
                poff = base + (k - 1) * GCHUNK
                pltpu.make_async_copy(
                    ras[nb], g_hbm.at[pl.ds(poff, GCHUNK), pl.ds(0, 16)],
                    sas[nb]).wait()
            pltpu.async_copy(a_hbm.at[ibs.at[k + 1]], ras[nb], gsa)
            pltpu.async_copy(b_hbm.at[ibd.at[k + 1]], rbs[nb], gsb)

        def add_row(i, carry, ra=ras[b], rb=rbs[b]):
            ra[i, :] = ra[i, :] + rb[i, :]
            return carry

        lax.fori_loop(0, GCHUNK, add_row, 0, unroll=4)
        pltpu.async_copy(ras[b],
                         g_hbm.at[pl.ds(off, GCHUNK), pl.ds(0, 16)], sas[b])
    for k in range(N_GCHUNKS - 2, N_GCHUNKS):
        b = k % 2
        off = base + k * GCHUNK
        pltpu.make_async_copy(
            ras[b], g_hbm.at[pl.ds(off, GCHUNK), pl.ds(0, 16)],
            sas[b]).wait()


def _sc_edge_gather(a_tab, b_tab, src, dst):
    fn = pl.kernel(
        _sc_edge_gather_body,
        out_type=jax.ShapeDtypeStruct((N_EDGES, 128), jnp.float32),
        mesh=_mesh,
        compiler_params=_sc_params,
        scratch_types=(
            pltpu.VMEM((N_GCHUNKS, GCHUNK), jnp.int32),
            pltpu.VMEM((N_GCHUNKS, GCHUNK), jnp.int32),
            pltpu.VMEM((GCHUNK, 16), jnp.float32),
            pltpu.VMEM((GCHUNK, 16), jnp.float32),
            pltpu.VMEM((GCHUNK, 16), jnp.float32),
            pltpu.VMEM((GCHUNK, 16), jnp.float32),
            pltpu.SemaphoreType.DMA,
            pltpu.SemaphoreType.DMA,
            pltpu.SemaphoreType.DMA,
            pltpu.SemaphoreType.DMA,
            pltpu.SemaphoreType.DMA,
        ),
    )
    return fn(a_tab, b_tab, src, dst)


def _tc_h1_body(nf_ref, w1_ref, degs_ref, degd_ref,
                h1p_ref, pq_ref):
    ds = degs_ref[...]
    dd = degd_ref[...]
    degs = ds[0, :N_NODES] + ds[1, :N_NODES]
    degd = dd[0, :N_NODES] + dd[1, :N_NODES]
    p = lax.rsqrt(jnp.maximum(degs, 1.0))
    q = lax.rsqrt(jnp.maximum(degd, 1.0))
    h1 = jnp.dot(nf_ref[...], w1_ref[...], preferred_element_type=jnp.float32)
    h1p_ref[...] = h1 * p[:, None]
    pq_ref[...] = jnp.concatenate([p[:, None], q[:, None]], axis=1)


def _tc_h1(n_feats, W1, degs, degd):
    return pl.pallas_call(
        _tc_h1_body,
        out_shape=(jax.ShapeDtypeStruct((N_NODES, 16), jnp.float32),
                   jax.ShapeDtypeStruct((N_NODES, 2), jnp.float32)),
    )(n_feats, W1, degs, degd)


def _tc_mid_body(p_ref, pq_ref, b1_ref, g1_ref, be1_ref, y_ref):
    parr = p_ref[...]
    agg = parr[0, :N_NODES, :] + parr[1, :N_NODES, :]
    q = pq_ref[:, 1]
    p = pq_ref[:, 0]
    s1 = g1_ref[...] * (1.0 / jnp.sqrt(1.0 + 1e-5))
    x1 = jnp.maximum(agg * q[:, None] + b1_ref[...][None, :], 0.0)
    y_ref[...] = (x1 * s1[None, :] + be1_ref[...][None, :]) * p[:, None]


def _tc_mid(partials, pq, b1, g1, be1):
    return pl.pallas_call(
        _tc_mid_body,
        out_shape=jax.ShapeDtypeStruct((N_NODES, 16), jnp.float32),
    )(partials, pq, b1, g1, be1)


def _tc_ab_body(p_ref, pq_ref, w2_ref, b2_ref, wp1_ref, sp_ref, wp2_ref,
                a_ref, b_ref):
    parr = p_ref[...]
    agg = parr[0, :N_NODES, :] + parr[1, :N_NODES, :]
    q = pq_ref[:, 1]
    aggq = agg * q[:, None]
    wp1 = wp1_ref[...]
    sp = sp_ref[...] * (1.0 / jnp.sqrt(1.0 + 1e-5))
    w2p = sp[:, None] * wp2_ref[...]
    ms = jnp.dot(wp1[16:80, :], w2p, preferred_element_type=jnp.float32)
    md = jnp.dot(wp1[80:144, :], w2p, preferred_element_type=jnp.float32)
    wa = jnp.dot(w2_ref[...], ms, preferred_element_type=jnp.float32)
    wb = jnp.dot(w2_ref[...], md, preferred_element_type=jnp.float32)
    ca = jnp.dot(b2_ref[...][None, :], ms, preferred_element_type=jnp.float32)
    cb = jnp.dot(b2_ref[...][None, :], md, preferred_element_type=jnp.float32)
    a = jnp.dot(aggq, wa, preferred_element_type=jnp.float32) + ca
    b = jnp.dot(aggq, wb, preferred_element_type=jnp.float32) + cb
    zpad = jnp.zeros((N_NODES, 6), dtype=jnp.float32)
    a_ref[...] = jnp.concatenate([a, zpad], axis=1)
    b_ref[...] = jnp.concatenate([b, zpad], axis=1)


def _tc_ab(partials, pq, W2, b2, Wp1, sp, Wp2):
    return pl.pallas_call(
        _tc_ab_body,
        out_shape=(jax.ShapeDtypeStruct((N_NODES, 16), jnp.float32),
                   jax.ShapeDtypeStruct((N_NODES, 16), jnp.float32)),
    )(partials, pq, W2, b2, Wp1, sp, Wp2)


_EBLK = 12800


def _tc_final_body(eft_ref, g_ref, wp1et_ref, wp2t_ref,
                   gp_ref, bepc_ref, bp1c_ref, bp2c_ref, out_ref):
    sp = gp_ref[...][0] * (1.0 / jnp.sqrt(1.0 + 1e-5))
    w2pt = wp2t_ref[...] * sp[None, :]
    met = jnp.dot(w2pt, wp1et_ref[...],
                  preferred_element_type=jnp.float32)
    c0t = (jnp.dot(w2pt, bp1c_ref[...],
                   preferred_element_type=jnp.float32)
           + jnp.dot(wp2t_ref[...], bepc_ref[...],
                     preferred_element_type=jnp.float32)
           + bp2c_ref[...])
    c0pt = jnp.concatenate(
        [c0t, jnp.full((6, 1), -1e30, jnp.float32)], axis=0)
    mept = jnp.concatenate(
        [met, jnp.zeros((6, 16), jnp.float32)], axis=0)
    zt = jnp.dot(mept, eft_ref[...], preferred_element_type=jnp.float32)
    gt = jnp.transpose(g_ref[...][:, :16])
    zt = zt + gt + c0pt
    m = jnp.max(zt, axis=0, keepdims=True)
    e = jnp.exp(zt - m)
    lse = jnp.log(jnp.sum(e, axis=0, keepdims=True))
    out_ref[...] = (zt - m - lse)[:10, :]


def _tc_final(eft, g, Wp1eT, Wp2T, gp, bepc, bp1c, bp2c):
    grid = N_EDGES // _EBLK
    return pl.pallas_call(
        _tc_final_body,
        grid=(grid,),
        in_specs=[
            pl.BlockSpec((16, _EBLK), lambda i: (0, i)),
            pl.BlockSpec((_EBLK, 128), lambda i: (i, 0)),
            pl.BlockSpec((16, 16), lambda i: (0, 0)),
            pl.BlockSpec((10, 16), lambda i: (0, 0)),
            pl.BlockSpec((1, 16), lambda i: (0, 0)),
            pl.BlockSpec((16, 1), lambda i: (0, 0)),
            pl.BlockSpec((16, 1), lambda i: (0, 0)),
            pl.BlockSpec((10, 1), lambda i: (0, 0)),
        ],
        out_specs=pl.BlockSpec((10, _EBLK), lambda i: (0, i)),
        out_shape=jax.ShapeDtypeStruct((10, N_EDGES), jnp.float32),
    )(eft, g, Wp1eT, Wp2T, gp, bepc, bp1c, bp2c)


@jax.jit
def _run(n_feats, edge_index, edge_feat, W1, b1, bn1_gamma, bn1_beta,
         W2, b2, Wp1, bp1, bnp_gamma, bnp_beta, Wp2, bp2):
    src = edge_index[0]
    dst = edge_index[1]
    ones = jnp.ones((CHUNK,), jnp.float32)
    zeros1 = jnp.zeros((N_PAD,), jnp.float32)
    zeros2 = jnp.zeros((N_PAD, 16), jnp.float32)

    degs, degd = _sc_degrees(src, dst, ones, zeros1)
    h1p, pq = _tc_h1(n_feats, W1, degs, degd)
    agg1 = _sc_agg(h1p, src, dst, zeros2)
    y = _tc_mid(agg1, pq, b1, bn1_gamma, bn1_beta)
    agg2 = _sc_agg(y, src, dst, zeros2)
    a_tab, b_tab = _tc_ab(agg2, pq, W2, b2, Wp1, bnp_gamma, Wp2)
    g = _sc_edge_gather(a_tab, b_tab, src, dst)
    outt = _tc_final(edge_feat.T, g, Wp1[:16, :].T, Wp2.T,
                     bnp_gamma[None, :], bnp_beta[:, None],
                     bp1[:, None], bp2[:, None])
    return outt.T


def kernel(n_feats, edge_index, edge_feat, W1, b1, bn1_gamma, bn1_beta,
           W2, b2, Wp1, bp1, bnp_gamma, bnp_beta, Wp2, bp2):
    return _run(n_feats, edge_index, edge_feat, W1, b1, bn1_gamma, bn1_beta,
                W2, b2, Wp1, bp1, bnp_gamma, bnp_beta, Wp2, bp2)


# speedup vs baseline: 1.0651x; 1.0096x over previous
import jax
import jax.numpy as jnp
from jax import lax
from jax.experimental import pallas as pl
from jax.experimental.pallas import tpu as pltpu
from jax.experimental.pallas import tpu_sc as plsc

N_NODES = 10000
N_PAD = 10240
N_EDGES = 320000
NC = 2
NS = 16
EDGES_PER_TILE = N_EDGES // (NC * NS)
CHUNK = 2000
N_CHUNKS = EDGES_PER_TILE // CHUNK
GCHUNK = 1000
N_GCHUNKS = EDGES_PER_TILE // GCHUNK
ROWS_PER_TILE = N_PAD // NS

_mesh = plsc.VectorSubcoreMesh(core_axis_name="c", subcore_axis_name="s")
_sc_params = pltpu.CompilerParams(use_tc_tiling_on_sc=False)


def _sc_degrees_body(src_hbm, dst_hbm, ones_hbm, zeros1_hbm,
                     degs_hbm, degd_hbm,
                     hist_s, hist_d, ibs, ibd, ones_v, obuf,
                     isem, ssem):
    c = lax.axis_index("c")
    s = lax.axis_index("s")
    wid = s * NC + c
    r0 = s * ROWS_PER_TILE
    base = wid * EDGES_PER_TILE
    for k in range(N_CHUNKS):
        off = base + k * CHUNK
        pltpu.async_copy(src_hbm.at[pl.ds(off, CHUNK)], ibs.at[k], isem)
        pltpu.async_copy(dst_hbm.at[pl.ds(off, CHUNK)], ibd.at[k], isem)
    pltpu.sync_copy(ones_hbm, ones_v)
    pltpu.sync_copy(zeros1_hbm.at[pl.ds(r0, ROWS_PER_TILE)],
                    hist_s.at[pl.ds(r0, ROWS_PER_TILE)])
    pltpu.sync_copy(zeros1_hbm.at[pl.ds(r0, ROWS_PER_TILE)],
                    hist_d.at[pl.ds(r0, ROWS_PER_TILE)])
    for k in range(N_CHUNKS):
        off = base + k * CHUNK
        pltpu.make_async_copy(src_hbm.at[pl.ds(off, CHUNK)],
                              ibs.at[k], isem).wait()
        pltpu.make_async_copy(dst_hbm.at[pl.ds(off, CHUNK)],
                              ibd.at[k], isem).wait()
    plsc.subcore_barrier()
    for k in range(N_CHUNKS):
        pltpu.async_copy(ones_v, hist_s.at[ibs.at[k]], ssem, add=True)
        pltpu.async_copy(ones_v, hist_d.at[ibd.at[k]], ssem, add=True)
    for k in range(N_CHUNKS):
        pltpu.make_async_copy(ones_v, hist_s.at[ibs.at[k]], ssem).wait()
        pltpu.make_async_copy(ones_v, hist_d.at[ibd.at[k]], ssem).wait()
    plsc.subcore_barrier()
    pltpu.sync_copy(hist_s.at[pl.ds(r0, ROWS_PER_TILE)], obuf)
    pltpu.sync_copy(obuf, degs_hbm.at[c, pl.ds(r0, ROWS_PER_TILE)])
    pltpu.sync_copy(hist_d.at[pl.ds(r0, ROWS_PER_TILE)], obuf)
    pltpu.sync_copy(obuf, degd_hbm.at[c, pl.ds(r0, ROWS_PER_TILE)])


def _sc_degrees(src, dst, ones, zeros1):
    fn = pl.kernel(
        _sc_degrees_body,
        out_type=(jax.ShapeDtypeStruct((NC, N_PAD), jnp.float32),
                  jax.ShapeDtypeStruct((NC, N_PAD), jnp.float32)),
        mesh=_mesh,
        compiler_params=_sc_params,
        scratch_types=(
            pltpu.VMEM_SHARED((N_PAD,), jnp.float32),
            pltpu.VMEM_SHARED((N_PAD,), jnp.float32),
            pltpu.VMEM((N_CHUNKS, CHUNK), jnp.int32),
            pltpu.VMEM((N_CHUNKS, CHUNK), jnp.int32),
            pltpu.VMEM((CHUNK,), jnp.float32),
            pltpu.VMEM((ROWS_PER_TILE,), jnp.float32),
            pltpu.SemaphoreType.DMA,
            pltpu.SemaphoreType.DMA,
        ),
    )
    return fn(src, dst, ones, zeros1)


def _sc_agg_body(tab_hbm, src_hbm, dst_hbm, zeros2_hbm,
                 out_hbm,
                 agg_sh, ibs, ibd, rb0, rb1, obuf,
                 isem, gsem, ss0, ss1):
    c = lax.axis_index("c")
    s = lax.axis_index("s")
    wid = s * NC + c
    r0 = s * ROWS_PER_TILE
    base = wid * EDGES_PER_TILE
    for k in range(N_CHUNKS):
        off = base + k * CHUNK
        pltpu.async_copy(src_hbm.at[pl.ds(off, CHUNK)], ibs.at[k], isem)
        pltpu.async_copy(dst_hbm.at[pl.ds(off, CHUNK)], ibd.at[k], isem)
    pltpu.sync_copy(zeros2_hbm.at[pl.ds(r0, ROWS_PER_TILE)],
                    agg_sh.at[pl.ds(r0, ROWS_PER_TILE)])
    for k in range(N_CHUNKS):
        off = base + k * CHUNK
        pltpu.make_async_copy(src_hbm.at[pl.ds(off, CHUNK)],
                              ibs.at[k], isem).wait()
        pltpu.make_async_copy(dst_hbm.at[pl.ds(off, CHUNK)],
                              ibd.at[k], isem).wait()
    plsc.subcore_barrier()
    rbs = (rb0, rb1)
    sss = (ss0, ss1)
    for k in range(N_CHUNKS):
        b = k % 2
        if k >= 2:
            pltpu.make_async_copy(rbs[b], agg_sh.at[ibd.at[k - 2]],
                                  sss[b]).wait()
        pltpu.async_copy(tab_hbm.at[ibs.at[k]], rbs[b], gsem)
        pltpu.make_async_copy(tab_hbm.at[ibs.at[k]], rbs[b], gsem).wait()
        pltpu.async_copy(rbs[b], agg_sh.at[ibd.at[k]], sss[b], add=True)
    for k in range(N_CHUNKS - 2, N_CHUNKS):
        b = k % 2
        pltpu.make_async_copy(rbs[b], agg_sh.at[ibd.at[k]], sss[b]).wait()
    plsc.subcore_barrier()
    pltpu.sync_copy(agg_sh.at[pl.ds(r0, ROWS_PER_TILE)], obuf)
    pltpu.sync_copy(obuf, out_hbm.at[c, pl.ds(r0, ROWS_PER_TILE)])


def _sc_agg(table, src, dst, zeros2):
    fn = pl.kernel(
        _sc_agg_body,
        out_type=jax.ShapeDtypeStruct((NC, N_PAD, 16), jnp.float32),
        mesh=_mesh,
        compiler_params=_sc_params,
        scratch_types=(
            pltpu.VMEM_SHARED((N_PAD, 16), jnp.float32),
            pltpu.VMEM((N_CHUNKS, CHUNK), jnp.int32),
            pltpu.VMEM((N_CHUNKS, CHUNK), jnp.int32),
            pltpu.VMEM((CHUNK, 16), jnp.float32),
            pltpu.VMEM((CHUNK, 16), jnp.float32),
            pltpu.VMEM((ROWS_PER_TILE, 16), jnp.float32),
            pltpu.SemaphoreType.DMA,
            pltpu.SemaphoreType.DMA,
            pltpu.SemaphoreType.DMA,
            pltpu.SemaphoreType.DMA,
        ),
    )
    return fn(table, src, dst, zeros2)


def _sc_edge_gather_body(a_hbm, b_hbm, src_hbm, dst_hbm,
                         g_hbm,
                         ibs, ibd, ra0, ra1, rb0, rb1,
                         isem, gsa, gsb, sa0, sa1):
    c = lax.axis_index("c")
    s = lax.axis_index("s")
    wid = s * NC + c
    base = wid * EDGES_PER_TILE
    for k in range(N_GCHUNKS):
        off = base + k * GCHUNK
        pltpu.async_copy(src_hbm.at[pl.ds(off, GCHUNK)], ibs.at[k], isem)
        pltpu.async_copy(dst_hbm.at[pl.ds(off, GCHUNK)], ibd.at[k], isem)
    for k in range(N_GCHUNKS):
        off = base + k * GCHUNK
        pltpu.make_async_copy(src_hbm.at[pl.ds(off, GCHUNK)],
                              ibs.at[k], isem).wait()
        pltpu.make_async_copy(dst_hbm.at[pl.ds(off, GCHUNK)],
                              ibd.at[k], isem).wait()
    ras = (ra0, ra1)
    rbs = (rb0, rb1)
    sas = (sa0, sa1)
    pltpu.async_copy(a_hbm.at[ibs.at[0]], ras[0], gsa)
    pltpu.async_copy(b_hbm.at[ibd.at[0]], rbs[0], gsb)
    for k in range(N_GCHUNKS):
        b = k % 2
        nb = (k + 1) % 2
        off = base + k * GCHUNK
        pltpu.make_async_copy(a_hbm.at[ibs.at[k]], ras[b], gsa).wait()
        pltpu.make_async_copy(b_hbm.at[ibd.at[k]], rbs[b], gsb).wait()
        if k + 1 < N_GCHUNKS:
            if k >= 1:
                poff = base + (k - 1) * GCHUNK
                pltpu.make_async_copy(
                    ras[nb], g_hbm.at[pl.ds(poff, GCHUNK), pl.ds(0, 16)],
                    sas[nb]).wait()
            pltpu.async_copy(a_hbm.at[ibs.at[k + 1]], ras[nb], gsa)
            pltpu.async_copy(b_hbm.at[ibd.at[k + 1]], rbs[nb], gsb)

        def add_row(i, carry, ra=ras[b], rb=rbs[b]):
            ra[i, :] = ra[i, :] + rb[i, :]
            return carry

        lax.fori_loop(0, GCHUNK, add_row, 0, unroll=4)
        pltpu.async_copy(ras[b],
                         g_hbm.at[pl.ds(off, GCHUNK), pl.ds(0, 16)], sas[b])
    for k in range(N_GCHUNKS - 2, N_GCHUNKS):
        b = k % 2
        off = base + k * GCHUNK
        pltpu.make_async_copy(
            ras[b], g_hbm.at[pl.ds(off, GCHUNK), pl.ds(0, 16)],
            sas[b]).wait()


def _sc_edge_gather(a_tab, b_tab, src, dst):
    fn = pl.kernel(
        _sc_edge_gather_body,
        out_type=jax.ShapeDtypeStruct((N_EDGES, 128), jnp.float32),
        mesh=_mesh,
        compiler_params=_sc_params,
        scratch_types=(
            pltpu.VMEM((N_GCHUNKS, GCHUNK), jnp.int32),
            pltpu.VMEM((N_GCHUNKS, GCHUNK), jnp.int32),
            pltpu.VMEM((GCHUNK, 16), jnp.float32),
            pltpu.VMEM((GCHUNK, 16), jnp.float32),
            pltpu.VMEM((GCHUNK, 16), jnp.float32),
            pltpu.VMEM((GCHUNK, 16), jnp.float32),
            pltpu.SemaphoreType.DMA,
            pltpu.SemaphoreType.DMA,
            pltpu.SemaphoreType.DMA,
            pltpu.SemaphoreType.DMA,
            pltpu.SemaphoreType.DMA,
        ),
    )
    return fn(a_tab, b_tab, src, dst)


def _tc_h1_body(nf_ref, w1_ref, degs_ref, degd_ref,
                h1p_ref, pq_ref):
    ds = degs_ref[...]
    dd = degd_ref[...]
    degs = ds[0, :N_NODES] + ds[1, :N_NODES]
    degd = dd[0, :N_NODES] + dd[1, :N_NODES]
    p = lax.rsqrt(jnp.maximum(degs, 1.0))
    q = lax.rsqrt(jnp.maximum(degd, 1.0))
    h1 = jnp.dot(nf_ref[...], w1_ref[...], preferred_element_type=jnp.float32)
    h1p_ref[...] = h1 * p[:, None]
    pq_ref[...] = jnp.concatenate([p[:, None], q[:, None]], axis=1)


def _tc_h1(n_feats, W1, degs, degd):
    return pl.pallas_call(
        _tc_h1_body,
        out_shape=(jax.ShapeDtypeStruct((N_NODES, 16), jnp.float32),
                   jax.ShapeDtypeStruct((N_NODES, 2), jnp.float32)),
    )(n_feats, W1, degs, degd)


def _tc_mid_body(p_ref, pq_ref, b1_ref, g1_ref, be1_ref, y_ref):
    parr = p_ref[...]
    agg = parr[0, :N_NODES, :] + parr[1, :N_NODES, :]
    q = pq_ref[:, 1]
    p = pq_ref[:, 0]
    s1 = g1_ref[...] * (1.0 / jnp.sqrt(1.0 + 1e-5))
    x1 = jnp.maximum(agg * q[:, None] + b1_ref[...][None, :], 0.0)
    y_ref[...] = (x1 * s1[None, :] + be1_ref[...][None, :]) * p[:, None]


def _tc_mid(partials, pq, b1, g1, be1):
    return pl.pallas_call(
        _tc_mid_body,
        out_shape=jax.ShapeDtypeStruct((N_NODES, 16), jnp.float32),
    )(partials, pq, b1, g1, be1)


def _tc_ab_body(p_ref, pq_ref, w2_ref, b2_ref, wp1_ref, sp_ref, wp2_ref,
                a_ref, b_ref):
    parr = p_ref[...]
    agg = parr[0, :N_NODES, :] + parr[1, :N_NODES, :]
    q = pq_ref[:, 1]
    aggq = agg * q[:, None]
    wp1 = wp1_ref[...]
    sp = sp_ref[...] * (1.0 / jnp.sqrt(1.0 + 1e-5))
    w2p = sp[:, None] * wp2_ref[...]
    ms = jnp.dot(wp1[16:80, :], w2p, preferred_element_type=jnp.float32)
    md = jnp.dot(wp1[80:144, :], w2p, preferred_element_type=jnp.float32)
    wa = jnp.dot(w2_ref[...], ms, preferred_element_type=jnp.float32)
    wb = jnp.dot(w2_ref[...], md, preferred_element_type=jnp.float32)
    ca = jnp.dot(b2_ref[...][None, :], ms, preferred_element_type=jnp.float32)
    cb = jnp.dot(b2_ref[...][None, :], md, preferred_element_type=jnp.float32)
    a = jnp.dot(aggq, wa, preferred_element_type=jnp.float32) + ca
    b = jnp.dot(aggq, wb, preferred_element_type=jnp.float32) + cb
    zpad = jnp.zeros((N_NODES, 6), dtype=jnp.float32)
    a_ref[...] = jnp.concatenate([a, zpad], axis=1)
    b_ref[...] = jnp.concatenate([b, zpad], axis=1)


def _tc_ab(partials, pq, W2, b2, Wp1, sp, Wp2):
    return pl.pallas_call(
        _tc_ab_body,
        out_shape=(jax.ShapeDtypeStruct((N_NODES, 16), jnp.float32),
                   jax.ShapeDtypeStruct((N_NODES, 16), jnp.float32)),
    )(partials, pq, W2, b2, Wp1, sp, Wp2)


_EBLK = 16000


def _tc_final_body(eft_ref, g_ref, wp1et_ref, wp2t_ref,
                   gp_ref, bepc_ref, bp1c_ref, bp2c_ref, out_ref):
    sp = gp_ref[...][0] * (1.0 / jnp.sqrt(1.0 + 1e-5))
    w2pt = wp2t_ref[...] * sp[None, :]
    met = jnp.dot(w2pt, wp1et_ref[...],
                  preferred_element_type=jnp.float32)
    c0t = (jnp.dot(w2pt, bp1c_ref[...],
                   preferred_element_type=jnp.float32)
           + jnp.dot(wp2t_ref[...], bepc_ref[...],
                     preferred_element_type=jnp.float32)
           + bp2c_ref[...])
    c0pt = jnp.concatenate(
        [c0t, jnp.full((6, 1), -1e30, jnp.float32)], axis=0)
    mept = jnp.concatenate(
        [met, jnp.zeros((6, 16), jnp.float32)], axis=0)
    zt = jnp.dot(mept, eft_ref[...], preferred_element_type=jnp.float32)
    gt = jnp.transpose(g_ref[...][:, :16])
    zt = zt + gt + c0pt
    m = jnp.max(zt, axis=0, keepdims=True)
    e = jnp.exp(zt - m)
    lse = jnp.log(jnp.sum(e, axis=0, keepdims=True))
    out_ref[...] = (zt - m - lse)[:10, :]


def _tc_final(eft, g, Wp1eT, Wp2T, gp, bepc, bp1c, bp2c):
    grid = N_EDGES // _EBLK
    return pl.pallas_call(
        _tc_final_body,
        grid=(grid,),
        in_specs=[
            pl.BlockSpec((16, _EBLK), lambda i: (0, i)),
            pl.BlockSpec((_EBLK, 128), lambda i: (i, 0)),
            pl.BlockSpec((16, 16), lambda i: (0, 0)),
            pl.BlockSpec((10, 16), lambda i: (0, 0)),
            pl.BlockSpec((1, 16), lambda i: (0, 0)),
            pl.BlockSpec((16, 1), lambda i: (0, 0)),
            pl.BlockSpec((16, 1), lambda i: (0, 0)),
            pl.BlockSpec((10, 1), lambda i: (0, 0)),
        ],
        out_specs=pl.BlockSpec((10, _EBLK), lambda i: (0, i)),
        out_shape=jax.ShapeDtypeStruct((10, N_EDGES), jnp.float32),
    )(eft, g, Wp1eT, Wp2T, gp, bepc, bp1c, bp2c)


@jax.jit
def _run(n_feats, edge_index, edge_feat, W1, b1, bn1_gamma, bn1_beta,
         W2, b2, Wp1, bp1, bnp_gamma, bnp_beta, Wp2, bp2):
    src = edge_index[0]
    dst = edge_index[1]
    ones = jnp.ones((CHUNK,), jnp.float32)
    zeros1 = jnp.zeros((N_PAD,), jnp.float32)
    zeros2 = jnp.zeros((N_PAD, 16), jnp.float32)

    degs, degd = _sc_degrees(src, dst, ones, zeros1)
    h1p, pq = _tc_h1(n_feats, W1, degs, degd)
    agg1 = _sc_agg(h1p, src, dst, zeros2)
    y = _tc_mid(agg1, pq, b1, bn1_gamma, bn1_beta)
    agg2 = _sc_agg(y, src, dst, zeros2)
    a_tab, b_tab = _tc_ab(agg2, pq, W2, b2, Wp1, bnp_gamma, Wp2)
    g = _sc_edge_gather(a_tab, b_tab, src, dst)
    outt = _tc_final(edge_feat.T, g, Wp1[:16, :].T, Wp2.T,
                     bnp_gamma[None, :], bnp_beta[:, None],
                     bp1[:, None], bp2[:, None])
    return outt.T


def kernel(n_feats, edge_index, edge_feat, W1, b1, bn1_gamma, bn1_beta,
           W2, b2, Wp1, bp1, bnp_gamma, bnp_beta, Wp2, bp2):
    return _run(n_feats, edge_index, edge_feat, W1, b1, bn1_gamma, bn1_beta,
                W2, b2, Wp1, bp1, bnp_gamma, bnp_beta, Wp2, bp2)
